# baseline (speedup 1.0000x reference)
import functools
import jax
import jax.numpy as jnp
from jax import lax
from jax.experimental import pallas as pl
from jax.experimental.pallas import tpu as pltpu
from jax.experimental.pallas import tpu_sc as plsc

D = 64
SCALE = 8.0
NC = 2
NS = 16
NW = NC * NS
LANES = 16
C = 128


def _make_kernel(n_rows: int):
  per_w = n_rows // NW
  G = per_w // C

  mesh = plsc.VectorSubcoreMesh(core_axis_name="c", subcore_axis_name="s")

  @functools.partial(
      pl.kernel,
      mesh=mesh,
      compiler_params=pltpu.CompilerParams(use_tc_tiling_on_sc=False),
      out_type=jax.ShapeDtypeStruct((n_rows, D), jnp.float32),
      scratch_types=[
          pltpu.VMEM((G, C), jnp.int32),
          pltpu.VMEM((C, D), jnp.float32),
      ],
  )
  def body(x_hbm, table_hbm, out_hbm, idx_v, rows_v):
    wid = lax.axis_index("s") * NC + lax.axis_index("c")
    base = wid * per_w
    pltpu.sync_copy(x_hbm.at[wid], idx_v)

    def chunk(g, carry):
      pltpu.sync_copy(table_hbm.at[idx_v.at[g]], rows_v)

      def row(i, c):
        for k in range(D // LANES):
          sl = pl.ds(k * LANES, LANES)
          rows_v[i, sl] = rows_v[i, sl] * SCALE
        return c

      lax.fori_loop(0, C, row, 0)

      pltpu.sync_copy(rows_v, out_hbm.at[pl.ds(base + g * C, C)])
      return carry

    lax.fori_loop(0, G, chunk, 0)

  return body


def kernel(x, table):
  B, L = x.shape
  n = B * L
  xw = x.astype(jnp.int32).reshape(NW, n // (NW * C), C)
  out = _make_kernel(n)(xw, table)
  return out.reshape(B, L, D)


# speedup vs baseline: 1.2101x; 1.2101x over previous
import functools
import jax
import jax.numpy as jnp
from jax import lax
from jax.experimental import pallas as pl
from jax.experimental.pallas import tpu as pltpu
from jax.experimental.pallas import tpu_sc as plsc

D = 64
SCALE = 8.0
NC = 2
NS = 16
NW = NC * NS
LANES = 16
C = 128


NB = 4


def _make_kernel(n_rows: int):
  per_w = n_rows // NW
  G = per_w // C
  assert G % NB == 0

  mesh = plsc.VectorSubcoreMesh(core_axis_name="c", subcore_axis_name="s")

  @functools.partial(
      pl.kernel,
      mesh=mesh,
      compiler_params=pltpu.CompilerParams(use_tc_tiling_on_sc=False),
      out_type=jax.ShapeDtypeStruct((n_rows, D), jnp.float32),
      scratch_types=[
          pltpu.VMEM((G, C), jnp.int32),
          pltpu.VMEM((NB, C, D), jnp.float32),
          pltpu.VMEM((NB, C, D), jnp.float32),
          [pltpu.SemaphoreType.DMA] * NB,
          [pltpu.SemaphoreType.DMA] * NB,
      ],
  )
  def body(x_hbm, table_hbm, out_hbm, idx_v, gbuf, sbuf, gsems, ssems):
    wid = lax.axis_index("s") * NC + lax.axis_index("c")
    base = wid * per_w
    pltpu.sync_copy(x_hbm.at[wid], idx_v)

    def start_gather(g, b):
      pltpu.async_copy(table_hbm.at[idx_v.at[g]], gbuf.at[b], gsems[b])

    for b in range(NB):
      start_gather(b, b)

    def outer(t, carry):
      for b in range(NB):
        g = t + b
        pltpu.make_async_copy(table_hbm.at[idx_v.at[g]], gbuf.at[b],
                              gsems[b]).wait()
        @pl.when(g >= NB)
        def _():
          pltpu.make_async_copy(
              sbuf.at[b], out_hbm.at[pl.ds(base + (g - NB) * C, C)],
              ssems[b]).wait()

        def row(i, c):
          for k in range(D // LANES):
            sl = pl.ds(k * LANES, LANES)
            sbuf[b, i, sl] = gbuf[b, i, sl] * SCALE
          return c

        lax.fori_loop(0, C, row, 0)

        pltpu.async_copy(sbuf.at[b], out_hbm.at[pl.ds(base + g * C, C)],
                         ssems[b])

        @pl.when(g + NB < G)
        def _():
          start_gather(g + NB, b)

      return carry

    lax.fori_loop(0, G // NB, lambda t, c: outer(t * NB, c), 0)

    for b in range(NB):
      g = G - NB + b
      pltpu.make_async_copy(sbuf.at[b], out_hbm.at[pl.ds(base + g * C, C)],
                            ssems[b]).wait()

  return body


def kernel(x, table):
  B, L = x.shape
  n = B * L
  xw = x.astype(jnp.int32).reshape(NW, n // (NW * C), C)
  out = _make_kernel(n)(xw, table)
  return out.reshape(B, L, D)
